# baseline (speedup 1.0000x reference)
import functools

import jax
import jax.numpy as jnp
from jax import lax
from jax.experimental import pallas as pl
from jax.experimental.pallas import tpu as pltpu
from jax.experimental.pallas import tpu_sc as plsc

_EMBED = 64
_SCALE = 8.0
_LANES = 16
_NUM_CORES = 2
_NUM_SUBCORES = 16
_NW = _NUM_CORES * _NUM_SUBCORES
_CHUNK = 128
_NBUF = 4


@functools.lru_cache(maxsize=None)
def _make_lookup(n_total: int):
    assert n_total % (_NW * _CHUNK) == 0
    per_w = n_total // _NW
    n_chunks = per_w // _CHUNK
    assert n_chunks >= 2 * _NBUF and n_chunks % _NBUF == 0

    mesh = plsc.VectorSubcoreMesh(
        core_axis_name="c", subcore_axis_name="s", num_cores=_NUM_CORES
    )

    @functools.partial(
        pl.kernel,
        mesh=mesh,
        compiler_params=pltpu.CompilerParams(needs_layout_passes=False),
        out_type=jax.ShapeDtypeStruct((n_total, _EMBED), jnp.float32),
        scratch_types=[
            pltpu.VMEM((n_chunks, _CHUNK), jnp.int32),
            *[pltpu.VMEM((_CHUNK,), jnp.int32) for _ in range(_NBUF)],
            *[pltpu.VMEM((_CHUNK, 2 * _EMBED), jnp.float32) for _ in range(_NBUF)],
            *[pltpu.VMEM((_CHUNK, _EMBED), jnp.float32) for _ in range(2)],
            *[pltpu.SemaphoreType.DMA for _ in range(_NBUF)],
            *[pltpu.SemaphoreType.DMA for _ in range(2)],
        ],
    )
    def lookup(idx_hbm, table_hbm, out_hbm, idx_v, *rest):
        pidx = rest[:_NBUF]
        bufs = rest[_NBUF : 2 * _NBUF]
        obufs = rest[2 * _NBUF : 2 * _NBUF + 2]
        gsems = rest[2 * _NBUF + 2 : 3 * _NBUF + 2]
        ssems = rest[3 * _NBUF + 2 :]
        wid = lax.axis_index("s") * _NUM_CORES + lax.axis_index("c")
        base = wid * per_w

        pltpu.sync_copy(idx_hbm.at[wid], idx_v)

        def start_gather(b, t):
            for g in range(_CHUNK // _LANES):
                sl = pl.ds(g * _LANES, _LANES)
                pidx[b][sl] = lax.shift_right_logical(idx_v[t, sl], 1)
            pltpu.async_copy(table_hbm.at[pidx[b]], bufs[b], gsems[b])

        def wait_gather(b):
            pltpu.make_async_copy(
                table_hbm.at[pidx[b]], bufs[b], gsems[b]
            ).wait()

        def select_scale(b, t, ob):
            buf = bufs[b]
            obuf = obufs[ob]
            lane = lax.iota(jnp.int32, _LANES)

            @pl.loop(0, _CHUNK // _LANES)
            def _(g):
                rows = g * _LANES + lane
                offv = (idx_v[t, pl.ds(g * _LANES, _LANES)] & 1) * _EMBED
                for c in range(_EMBED):
                    v = plsc.load_gather(buf, [rows, offv + c])
                    plsc.store_scatter(obuf, [rows, jnp.full((_LANES,), c, jnp.int32)], v * _SCALE)

        def start_store(ob, t):
            pltpu.async_copy(
                obufs[ob], out_hbm.at[pl.ds(base + t * _CHUNK, _CHUNK)], ssems[ob]
            )

        def wait_store(ob, t):
            pltpu.make_async_copy(
                obufs[ob], out_hbm.at[pl.ds(base + t * _CHUNK, _CHUNK)], ssems[ob]
            ).wait()

        for b in range(_NBUF):
            start_gather(b, b)

        @pl.loop(0, n_chunks - _NBUF, step=_NBUF)
        def _(cbase):
            for b in range(_NBUF):
                t = cbase + b
                ob = b % 2
                wait_gather(b)
                @pl.when(t >= 2)
                def _():
                    wait_store(ob, t - 2)
                select_scale(b, t, ob)
                start_store(ob, t)
                start_gather(b, t + _NBUF)

        for b in range(_NBUF):
            t = n_chunks - _NBUF + b
            ob = b % 2
            wait_gather(b)
            wait_store(ob, t - 2)
            select_scale(b, t, ob)
            start_store(ob, t)
        wait_store((n_chunks - 2) % 2, n_chunks - 2)
        wait_store((n_chunks - 1) % 2, n_chunks - 1)

    return lookup


def kernel(x, embedding):
    batch, seq = x.shape
    n_total = batch * seq
    vocab, embed = embedding.shape
    idx = x.reshape(_NW, n_total // (_NW * _CHUNK), _CHUNK).astype(jnp.int32)
    table2 = embedding.reshape(vocab // 2, 2 * embed)
    out = _make_lookup(n_total)(idx, table2)
    return out.reshape(batch, seq, _EMBED)


# speedup vs baseline: 2.2424x; 2.2424x over previous
import functools

import jax
import jax.numpy as jnp
from jax import lax
from jax.experimental import pallas as pl
from jax.experimental.pallas import tpu as pltpu
from jax.experimental.pallas import tpu_sc as plsc

_EMBED = 64
_HALF = 32
_SCALE = 8.0
_LANES = 16
_NUM_CORES = 2
_NUM_SUBCORES = 16
_NW = _NUM_CORES * _NUM_SUBCORES
_CHUNK = 128
_NBUF = 4


@functools.lru_cache(maxsize=None)
def _make_lookup(n_total: int):
    assert n_total % (_NW * _CHUNK) == 0
    per_w = n_total // _NW
    n_chunks = per_w // _CHUNK
    assert n_chunks >= 2 * _NBUF and n_chunks % _NBUF == 0

    mesh = plsc.VectorSubcoreMesh(
        core_axis_name="c", subcore_axis_name="s", num_cores=_NUM_CORES
    )

    @functools.partial(
        pl.kernel,
        mesh=mesh,
        compiler_params=pltpu.CompilerParams(
            needs_layout_passes=False, use_tc_tiling_on_sc=False
        ),
        out_type=jax.ShapeDtypeStruct((2 * n_total, _HALF), jnp.float32),
        scratch_types=[
            pltpu.VMEM((n_chunks, _CHUNK), jnp.int32),
            *[pltpu.VMEM((2 * _CHUNK,), jnp.int32) for _ in range(_NBUF)],
            *[pltpu.VMEM((2 * _CHUNK, _HALF), jnp.float32) for _ in range(_NBUF)],
            *[pltpu.SemaphoreType.DMA for _ in range(2 * _NBUF)],
            *[pltpu.SemaphoreType.DMA for _ in range(_NBUF)],
        ],
    )
    def lookup(idx_hbm, table_hbm, out_hbm, idx_v, *rest):
        pidx = rest[:_NBUF]
        bufs = rest[_NBUF : 2 * _NBUF]
        gsems = rest[2 * _NBUF : 4 * _NBUF]
        ssems = rest[4 * _NBUF :]
        wid = lax.axis_index("s") * _NUM_CORES + lax.axis_index("c")
        base = wid * per_w

        pltpu.sync_copy(idx_hbm.at[wid], idx_v)

        lane = lax.iota(jnp.int32, _LANES)
        lane2 = lane + lane

        def start_gather(b, t):
            for g in range(_CHUNK // _LANES):
                xv = idx_v[t, pl.ds(g * _LANES, _LANES)]
                ev = xv + xv
                pos = 2 * g * _LANES + lane2
                plsc.store_scatter(pidx[b], [pos], ev)
                plsc.store_scatter(pidx[b], [pos + 1], ev + 1)
            for h in range(2):
                sl = pl.ds(h * _CHUNK, _CHUNK)
                pltpu.async_copy(
                    table_hbm.at[pidx[b].at[sl]], bufs[b].at[sl], gsems[2 * b + h]
                )

        def wait_gather(b):
            for h in range(2):
                sl = pl.ds(h * _CHUNK, _CHUNK)
                pltpu.make_async_copy(
                    table_hbm.at[pidx[b].at[sl]], bufs[b].at[sl], gsems[2 * b + h]
                ).wait()

        def scale(b):
            buf = bufs[b]

            @pl.loop(0, 2 * _CHUNK // 8)
            def _(g):
                for r in range(8):
                    for c in range(_HALF // _LANES):
                        sl = pl.ds(c * _LANES, _LANES)
                        buf[g * 8 + r, sl] = buf[g * 8 + r, sl] * _SCALE

        def start_store(b, t):
            pltpu.async_copy(
                bufs[b],
                out_hbm.at[pl.ds(2 * (base + t * _CHUNK), 2 * _CHUNK)],
                ssems[b],
            )

        def wait_store(b, t):
            pltpu.make_async_copy(
                bufs[b],
                out_hbm.at[pl.ds(2 * (base + t * _CHUNK), 2 * _CHUNK)],
                ssems[b],
            ).wait()

        for b in range(_NBUF):
            start_gather(b, b)

        @pl.loop(0, n_chunks - _NBUF, step=_NBUF)
        def _(cbase):
            for b in range(_NBUF):
                t = cbase + b
                wait_gather(b)
                scale(b)
                start_store(b, t)
                wait_store(b, t)
                start_gather(b, t + _NBUF)

        for b in range(_NBUF):
            t = n_chunks - _NBUF + b
            wait_gather(b)
            scale(b)
            start_store(b, t)
            wait_store(b, t)

    return lookup


def kernel(x, embedding):
    batch, seq = x.shape
    n_total = batch * seq
    vocab, embed = embedding.shape
    idx = x.reshape(_NW, n_total // (_NW * _CHUNK), _CHUNK).astype(jnp.int32)
    table2 = embedding.reshape(2 * vocab, _HALF)
    out = _make_lookup(n_total)(idx, table2)
    return out.reshape(batch, seq, _EMBED)
